# baseline (speedup 1.0000x reference)
import functools

import jax
import jax.numpy as jnp
from jax import lax
from jax.experimental import pallas as pl
from jax.experimental.pallas import tpu as pltpu
from jax.experimental.pallas import tpu_sc as plsc

NN = 10000
NE = 320000
F = 128
NG = 64
NCLS = 40
NS = 16
CHUNK = 128
CPS = 160
EPAD = NS * CPS * CHUNK
HALF = 5120
NROWH = 5248
RPTH = NROWH // NS
NBUF = 4
SPC = CPS // 2

_mesh = plsc.VectorSubcoreMesh(core_axis_name="c", subcore_axis_name="s")


def _zero_acc(rows, acc_sh, s):
    zero_row = jnp.zeros((16,), jnp.float32)

    def init_zero(i, carry):
        for j in range(F // 16):
            rows[i, pl.ds(j * 16, 16)] = zero_row
        return carry

    lax.fori_loop(0, CHUNK, init_zero, 0)
    for off, n in ((0, CHUNK), (CHUNK, CHUNK), (2 * CHUNK, RPTH - 2 * CHUNK)):
        pltpu.sync_copy(rows.at[pl.ds(0, n)],
                        acc_sh.at[pl.ds(s * RPTH + off, n)])


def _copy_out(rows, acc_sh, out_hbm, c, s):
    for off, n in ((0, CHUNK), (CHUNK, CHUNK), (2 * CHUNK, RPTH - 2 * CHUNK)):
        pltpu.sync_copy(acc_sh.at[pl.ds(s * RPTH + off, n)],
                        rows.at[pl.ds(0, n)])
        pltpu.sync_copy(rows.at[pl.ds(0, n)],
                        out_hbm.at[c, pl.ds(s * RPTH + off, n)])


def _chunks_this_stage(nbuf_chunks, stage):
    nst = jnp.clip(nbuf_chunks - stage * SPC, 0, SPC)
    return (nst + (NBUF - 1)) // NBUF


@functools.partial(
    pl.kernel,
    out_type=[
        jax.ShapeDtypeStruct((2, NS, CPS, CHUNK), jnp.int32),
        jax.ShapeDtypeStruct((2, NS, CPS, CHUNK), jnp.int32),
        jax.ShapeDtypeStruct((2, NS, 8, CHUNK), jnp.int32),
    ],
    mesh=_mesh,
    compiler_params=pltpu.CompilerParams(needs_layout_passes=False),
    scratch_types=[
        pltpu.VMEM((CPS, CHUNK), jnp.int32),
        pltpu.VMEM((CPS, CHUNK), jnp.int32),
        pltpu.VMEM((CPS, CHUNK), jnp.int32),
        pltpu.VMEM((CPS, CHUNK), jnp.int32),
        pltpu.VMEM((8, CHUNK), jnp.int32),
    ],
)
def _part_kernel(src_hbm, dst_hbm, srcl_hbm, dstl_hbm, cnt_hbm,
                 sstage, dstage, srcl, dstl, cntb):
    c = lax.axis_index("c")
    s = lax.axis_index("s")
    pltpu.sync_copy(src_hbm.at[s], sstage)
    pltpu.sync_copy(dst_hbm.at[s], dstage)

    zero16 = jnp.zeros((16,), jnp.int32)
    sink16 = zero16 + HALF

    def init_lists(t, carry):
        for j in range(CHUNK // 16):
            srcl[t, pl.ds(j * 16, 16)] = zero16
            dstl[t, pl.ds(j * 16, 16)] = sink16
        return carry

    lax.fori_loop(0, CPS, init_lists, 0)

    base = c * HALF

    def scan(t, ptr):
        for j in range(CHUNK // 16):
            vd = dstage[t, pl.ds(j * 16, 16)]
            vs = sstage[t, pl.ds(j * 16, 16)]
            rel = vd - base
            ok = (rel >= 0) & (rel < HALF)
            pos = ptr + plsc.cumsum(jnp.where(ok, 1, 0)) - 1
            row = pos >> 7
            col = pos & 127
            plsc.store_scatter(dstl, [row, col], rel, mask=ok)
            plsc.store_scatter(srcl, [row, col], vs, mask=ok)
            ptr = ptr + plsc.all_reduce_population_count(ok)
        return ptr

    ptr = lax.fori_loop(0, CPS, scan, jnp.zeros((16,), jnp.int32))

    for a in range(8):
        for j in range(CHUNK // 16):
            cntb[a, pl.ds(j * 16, 16)] = ptr

    pltpu.sync_copy(srcl, srcl_hbm.at[c, s])
    pltpu.sync_copy(dstl, dstl_hbm.at[c, s])
    pltpu.sync_copy(cntb, cnt_hbm.at[c, s])


@functools.partial(
    pl.kernel,
    out_type=jax.ShapeDtypeStruct((2, NROWH, F), jnp.float32),
    mesh=_mesh,
    scratch_types=[
        pltpu.VMEM((SPC, CHUNK), jnp.int32),
        pltpu.VMEM((SPC, CHUNK), jnp.int32),
        pltpu.VMEM((8, CHUNK), jnp.int32),
        [pltpu.VMEM((CHUNK, F), jnp.float32)] * NBUF,
        [pltpu.SemaphoreType.DMA] * NBUF,
        pltpu.VMEM_SHARED((NROWH, F), jnp.float32),
    ],
)
def _prop_kernel(g_hbm, srcl_hbm, dstl_hbm, cnt_hbm, out_hbm,
                 sidx, didx, cntb, rows, sems, acc_sh):
    c = lax.axis_index("c")
    s = lax.axis_index("s")
    _zero_acc(rows[0], acc_sh, s)
    plsc.subcore_barrier()

    pltpu.sync_copy(cnt_hbm.at[c, s], cntb)
    n_edges = cntb[0, pl.ds(0, 16)][0]
    n_chunks = (n_edges + (CHUNK - 1)) // CHUNK

    for stage in range(2):
        pltpu.sync_copy(srcl_hbm.at[c, s, pl.ds(stage * SPC, SPC)], sidx)
        pltpu.sync_copy(dstl_hbm.at[c, s, pl.ds(stage * SPC, SPC)], didx)
        for b in range(NBUF):
            pltpu.async_copy(g_hbm.at[sidx.at[b]], rows[b], sems[b])

        def body(i, carry):
            for b in range(NBUF):
                t = i * NBUF + b
                pltpu.make_async_copy(g_hbm.at[sidx.at[t]], rows[b],
                                      sems[b]).wait()
                pltpu.sync_copy(rows[b], acc_sh.at[didx.at[t]], add=True)
                tn = jnp.minimum(t + NBUF, SPC - 1)
                pltpu.async_copy(g_hbm.at[sidx.at[tn]], rows[b], sems[b])
            return carry

        lax.fori_loop(0, SPC // NBUF, body, 0)
        for b in range(NBUF):
            pltpu.make_async_copy(g_hbm.at[sidx.at[SPC - 1]], rows[b],
                                  sems[b]).wait()
    plsc.subcore_barrier()
    _copy_out(rows[0], acc_sh, out_hbm, c, s)


@functools.partial(
    pl.kernel,
    out_type=jax.ShapeDtypeStruct((2, NROWH, F), jnp.float32),
    mesh=_mesh,
    scratch_types=[
        pltpu.VMEM((SPC, CHUNK), jnp.int32),
        pltpu.VMEM((8, CHUNK), jnp.int32),
        pltpu.VMEM((CHUNK, F), jnp.float32),
        pltpu.VMEM_SHARED((NROWH, F), jnp.float32),
    ],
)
def _deg_kernel(dstl_hbm, cnt_hbm, out_hbm, didx, cntb, rows, acc_sh):
    c = lax.axis_index("c")
    s = lax.axis_index("s")
    _zero_acc(rows, acc_sh, s)
    plsc.subcore_barrier()

    pltpu.sync_copy(cnt_hbm.at[c, s], cntb)
    n_edges = cntb[0, pl.ds(0, 16)][0]
    n_chunks = (n_edges + (CHUNK - 1)) // CHUNK

    one_row = jnp.zeros((16,), jnp.float32) + 1.0

    def init_ones(i, carry):
        for j in range(F // 16):
            rows[i, pl.ds(j * 16, 16)] = one_row
        return carry

    lax.fori_loop(0, CHUNK, init_ones, 0)

    for stage in range(2):
        pltpu.sync_copy(dstl_hbm.at[c, s, pl.ds(stage * SPC, SPC)], didx)

        def body(i, carry):
            for b in range(NBUF):
                t = i * NBUF + b
                pltpu.sync_copy(rows, acc_sh.at[didx.at[t]], add=True)
            return carry

        lax.fori_loop(0, _chunks_this_stage(n_chunks, stage), body, 0)
    plsc.subcore_barrier()
    _copy_out(rows, acc_sh, out_hbm, c, s)


def _tc1_body(x_ref, w_ref, degp_ref, g_ref, dinv_ref):
    d = jnp.concatenate(
        [degp_ref[0, :HALF, 0:1], degp_ref[1, :NN - HALF, 0:1]],
        axis=0) + 1.0
    dinv = lax.rsqrt(d)
    dinv_ref[...] = dinv
    h = jnp.dot(x_ref[...], w_ref[...], preferred_element_type=jnp.float32)
    g_ref[...] = dinv * h


_tc1 = pl.pallas_call(
    _tc1_body,
    out_shape=[
        jax.ShapeDtypeStruct((NN, F), jnp.float32),
        jax.ShapeDtypeStruct((NN, 1), jnp.float32),
    ],
)


def _agg(p_ref, g_ref):
    return jnp.concatenate(
        [p_ref[0, :HALF, :], p_ref[1, :NN - HALF, :]], axis=0) + g_ref[...]


def _tc_mid_body(p_ref, g_ref, dinv_ref, b_ref, w_ref, out_ref):
    xn = jnp.maximum(dinv_ref[...] * _agg(p_ref, g_ref) + b_ref[...], 0.0)
    h = jnp.dot(xn, w_ref[...], preferred_element_type=jnp.float32)
    out_ref[...] = dinv_ref[...] * h


_tc_mid = pl.pallas_call(
    _tc_mid_body,
    out_shape=jax.ShapeDtypeStruct((NN, F), jnp.float32),
)


def _tc_fin_body(p_ref, g_ref, dinv_ref, b_ref, batch_ref, wl_ref, bl_ref,
                 out_ref):
    x4 = dinv_ref[...] * _agg(p_ref, g_ref) + b_ref[...]
    gid = lax.broadcasted_iota(jnp.int32, (NG, NN), 0)
    m = (batch_ref[...] == gid).astype(jnp.float32)
    sums = jnp.dot(m, x4, preferred_element_type=jnp.float32)
    counts = jnp.sum(m, axis=1, keepdims=True)
    pooled = sums / jnp.maximum(counts, 1.0)
    out_ref[...] = (
        jnp.dot(pooled, wl_ref[...], preferred_element_type=jnp.float32)
        + bl_ref[...]
    )


_tc_fin = pl.pallas_call(
    _tc_fin_body,
    out_shape=jax.ShapeDtypeStruct((NG, F), jnp.float32),
)


def kernel(x, edge_index, batch, W1, b1, W2, b2, W3, b3, Wl, bl):
    src = edge_index[0].astype(jnp.int32)
    dst = edge_index[1].astype(jnp.int32)
    pad = EPAD - NE
    srcp = jnp.concatenate([src, jnp.zeros((pad,), jnp.int32)])
    srcp = srcp.reshape(NS, CPS, CHUNK)
    dstp = jnp.concatenate([dst, jnp.full((pad,), NN, jnp.int32)])
    dstp = dstp.reshape(NS, CPS, CHUNK)

    srcl, dstl, cnt = _part_kernel(srcp, dstp)
    degp = _deg_kernel(dstl, cnt)
    g1, dinv = _tc1(x, W1, degp)
    p1 = _prop_kernel(g1, srcl, dstl, cnt)
    g2 = _tc_mid(p1, g1, dinv, b1.reshape(1, F), W2)
    p2 = _prop_kernel(g2, srcl, dstl, cnt)
    g3 = _tc_mid(p2, g2, dinv, b2.reshape(1, F), W3)
    p3 = _prop_kernel(g3, srcl, dstl, cnt)

    wlp = jnp.pad(Wl, ((0, 0), (0, F - NCLS)))
    blp = jnp.pad(bl, (0, F - NCLS)).reshape(1, F)
    out = _tc_fin(p3, g3, dinv, b3.reshape(1, F),
                  batch.astype(jnp.int32).reshape(1, NN), wlp, blp)
    return out[:, :NCLS]


# speedup vs baseline: 24.6644x; 24.6644x over previous
import functools

import jax
import jax.numpy as jnp
from jax import lax
from jax.experimental import pallas as pl
from jax.experimental.pallas import tpu as pltpu
from jax.experimental.pallas import tpu_sc as plsc

NN = 10000
NE = 320000
F = 128
NG = 64
NCLS = 40
NS = 16
CHUNK = 128
CPS = 160
EPAD = NS * CPS * CHUNK
HALF = 5120
NROWH = 5248
RPTH = NROWH // NS
NBUF = 4
SPC = CPS // 2

_mesh = plsc.VectorSubcoreMesh(core_axis_name="c", subcore_axis_name="s")


def _zero_acc(rows, acc_sh, s):
    zero_row = jnp.zeros((16,), jnp.float32)

    def init_zero(i, carry):
        for j in range(F // 16):
            rows[i, pl.ds(j * 16, 16)] = zero_row
        return carry

    lax.fori_loop(0, CHUNK, init_zero, 0)
    for off, n in ((0, CHUNK), (CHUNK, CHUNK), (2 * CHUNK, RPTH - 2 * CHUNK)):
        pltpu.sync_copy(rows.at[pl.ds(0, n)],
                        acc_sh.at[pl.ds(s * RPTH + off, n)])


def _copy_out(rows, acc_sh, out_hbm, c, s):
    for off, n in ((0, CHUNK), (CHUNK, CHUNK), (2 * CHUNK, RPTH - 2 * CHUNK)):
        pltpu.sync_copy(acc_sh.at[pl.ds(s * RPTH + off, n)],
                        rows.at[pl.ds(0, n)])
        pltpu.sync_copy(rows.at[pl.ds(0, n)],
                        out_hbm.at[c, pl.ds(s * RPTH + off, n)])


def _chunks_this_stage(nbuf_chunks, stage):
    nst = jnp.clip(nbuf_chunks - stage * SPC, 0, SPC)
    return (nst + (NBUF - 1)) // NBUF


@functools.partial(
    pl.kernel,
    out_type=[
        jax.ShapeDtypeStruct((2, NS, CPS, CHUNK), jnp.int32),
        jax.ShapeDtypeStruct((2, NS, CPS, CHUNK), jnp.int32),
        jax.ShapeDtypeStruct((2, NS, 8, CHUNK), jnp.int32),
    ],
    mesh=_mesh,
    compiler_params=pltpu.CompilerParams(needs_layout_passes=False),
    scratch_types=[
        pltpu.VMEM((CPS, CHUNK), jnp.int32),
        pltpu.VMEM((CPS, CHUNK), jnp.int32),
        pltpu.VMEM((CPS, CHUNK), jnp.int32),
        pltpu.VMEM((CPS, CHUNK), jnp.int32),
        pltpu.VMEM((8, CHUNK), jnp.int32),
    ],
)
def _part_kernel(src_hbm, dst_hbm, srcl_hbm, dstl_hbm, cnt_hbm,
                 sstage, dstage, srcl, dstl, cntb):
    c = lax.axis_index("c")
    s = lax.axis_index("s")
    pltpu.sync_copy(src_hbm.at[s], sstage)
    pltpu.sync_copy(dst_hbm.at[s], dstage)

    lane = lax.iota(jnp.int32, 16)

    def init_lists(t, carry):
        for j in range(CHUNK // 16):
            srcl[t, pl.ds(j * 16, 16)] = lane + (j * 16)
            dstl[t, pl.ds(j * 16, 16)] = lane + (HALF + j * 16)
        return carry

    lax.fori_loop(0, CPS, init_lists, 0)

    base = c * HALF

    def scan(t, ptr):
        for j in range(CHUNK // 16):
            vd = dstage[t, pl.ds(j * 16, 16)]
            vs = sstage[t, pl.ds(j * 16, 16)]
            rel = vd - base
            ok = (rel >= 0) & (rel < HALF)
            pos = ptr + plsc.cumsum(jnp.where(ok, 1, 0)) - 1
            row = pos >> 7
            col = pos & 127
            plsc.store_scatter(dstl, [row, col], rel, mask=ok)
            plsc.store_scatter(srcl, [row, col], vs, mask=ok)
            ptr = ptr + plsc.all_reduce_population_count(ok)
        return ptr

    ptr = lax.fori_loop(0, CPS, scan, jnp.zeros((16,), jnp.int32))

    for a in range(8):
        for j in range(CHUNK // 16):
            cntb[a, pl.ds(j * 16, 16)] = ptr

    pltpu.sync_copy(srcl, srcl_hbm.at[c, s])
    pltpu.sync_copy(dstl, dstl_hbm.at[c, s])
    pltpu.sync_copy(cntb, cnt_hbm.at[c, s])


@functools.partial(
    pl.kernel,
    out_type=jax.ShapeDtypeStruct((2, NROWH, F), jnp.float32),
    mesh=_mesh,
    scratch_types=[
        pltpu.VMEM((SPC, CHUNK), jnp.int32),
        pltpu.VMEM((SPC, CHUNK), jnp.int32),
        pltpu.VMEM((8, CHUNK), jnp.int32),
        [pltpu.VMEM((CHUNK, F), jnp.float32)] * NBUF,
        [pltpu.SemaphoreType.DMA] * NBUF,
        pltpu.VMEM_SHARED((NROWH, F), jnp.float32),
    ],
)
def _prop_kernel(g_hbm, srcl_hbm, dstl_hbm, cnt_hbm, out_hbm,
                 sidx, didx, cntb, rows, sems, acc_sh):
    c = lax.axis_index("c")
    s = lax.axis_index("s")
    _zero_acc(rows[0], acc_sh, s)
    plsc.subcore_barrier()

    pltpu.sync_copy(cnt_hbm.at[c, s], cntb)
    n_edges = cntb[0, pl.ds(0, 16)][0]
    n_chunks = (n_edges + (CHUNK - 1)) // CHUNK

    for stage in range(2):
        pltpu.sync_copy(srcl_hbm.at[c, s, pl.ds(stage * SPC, SPC)], sidx)
        pltpu.sync_copy(dstl_hbm.at[c, s, pl.ds(stage * SPC, SPC)], didx)
        for b in range(NBUF):
            pltpu.async_copy(g_hbm.at[sidx.at[b]], rows[b], sems[b])

        def body(i, carry):
            for b in range(NBUF):
                t = i * NBUF + b
                pltpu.make_async_copy(g_hbm.at[sidx.at[t]], rows[b],
                                      sems[b]).wait()
                pltpu.sync_copy(rows[b], acc_sh.at[didx.at[t]], add=True)
                tn = jnp.minimum(t + NBUF, SPC - 1)
                pltpu.async_copy(g_hbm.at[sidx.at[tn]], rows[b], sems[b])
            return carry

        lax.fori_loop(0, _chunks_this_stage(n_chunks, stage), body, 0)
        for b in range(NBUF):
            pltpu.make_async_copy(g_hbm.at[sidx.at[SPC - 1]], rows[b],
                                  sems[b]).wait()
    plsc.subcore_barrier()
    _copy_out(rows[0], acc_sh, out_hbm, c, s)


@functools.partial(
    pl.kernel,
    out_type=jax.ShapeDtypeStruct((2, NROWH, F), jnp.float32),
    mesh=_mesh,
    scratch_types=[
        pltpu.VMEM((SPC, CHUNK), jnp.int32),
        pltpu.VMEM((8, CHUNK), jnp.int32),
        pltpu.VMEM((CHUNK, F), jnp.float32),
        pltpu.VMEM_SHARED((NROWH, F), jnp.float32),
    ],
)
def _deg_kernel(dstl_hbm, cnt_hbm, out_hbm, didx, cntb, rows, acc_sh):
    c = lax.axis_index("c")
    s = lax.axis_index("s")
    _zero_acc(rows, acc_sh, s)
    plsc.subcore_barrier()

    pltpu.sync_copy(cnt_hbm.at[c, s], cntb)
    n_edges = cntb[0, pl.ds(0, 16)][0]
    n_chunks = (n_edges + (CHUNK - 1)) // CHUNK

    one_row = jnp.zeros((16,), jnp.float32) + 1.0

    def init_ones(i, carry):
        for j in range(F // 16):
            rows[i, pl.ds(j * 16, 16)] = one_row
        return carry

    lax.fori_loop(0, CHUNK, init_ones, 0)

    for stage in range(2):
        pltpu.sync_copy(dstl_hbm.at[c, s, pl.ds(stage * SPC, SPC)], didx)

        def body(i, carry):
            for b in range(NBUF):
                t = i * NBUF + b
                pltpu.sync_copy(rows, acc_sh.at[didx.at[t]], add=True)
            return carry

        lax.fori_loop(0, _chunks_this_stage(n_chunks, stage), body, 0)
    plsc.subcore_barrier()
    _copy_out(rows, acc_sh, out_hbm, c, s)


def _tc1_body(x_ref, w_ref, degp_ref, g_ref, dinv_ref):
    d = jnp.concatenate(
        [degp_ref[0, :HALF, 0:1], degp_ref[1, :NN - HALF, 0:1]],
        axis=0) + 1.0
    dinv = lax.rsqrt(d)
    dinv_ref[...] = dinv
    h = jnp.dot(x_ref[...], w_ref[...], preferred_element_type=jnp.float32)
    g_ref[...] = dinv * h


_tc1 = pl.pallas_call(
    _tc1_body,
    out_shape=[
        jax.ShapeDtypeStruct((NN, F), jnp.float32),
        jax.ShapeDtypeStruct((NN, 1), jnp.float32),
    ],
)


def _agg(p_ref, g_ref):
    return jnp.concatenate(
        [p_ref[0, :HALF, :], p_ref[1, :NN - HALF, :]], axis=0) + g_ref[...]


def _tc_mid_body(p_ref, g_ref, dinv_ref, b_ref, w_ref, out_ref):
    xn = jnp.maximum(dinv_ref[...] * _agg(p_ref, g_ref) + b_ref[...], 0.0)
    h = jnp.dot(xn, w_ref[...], preferred_element_type=jnp.float32)
    out_ref[...] = dinv_ref[...] * h


_tc_mid = pl.pallas_call(
    _tc_mid_body,
    out_shape=jax.ShapeDtypeStruct((NN, F), jnp.float32),
)


def _tc_fin_body(p_ref, g_ref, dinv_ref, b_ref, batch_ref, wl_ref, bl_ref,
                 out_ref):
    x4 = dinv_ref[...] * _agg(p_ref, g_ref) + b_ref[...]
    gid = lax.broadcasted_iota(jnp.int32, (NG, NN), 0)
    m = (batch_ref[...] == gid).astype(jnp.float32)
    sums = jnp.dot(m, x4, preferred_element_type=jnp.float32)
    counts = jnp.sum(m, axis=1, keepdims=True)
    pooled = sums / jnp.maximum(counts, 1.0)
    out_ref[...] = (
        jnp.dot(pooled, wl_ref[...], preferred_element_type=jnp.float32)
        + bl_ref[...]
    )


_tc_fin = pl.pallas_call(
    _tc_fin_body,
    out_shape=jax.ShapeDtypeStruct((NG, F), jnp.float32),
)


def kernel(x, edge_index, batch, W1, b1, W2, b2, W3, b3, Wl, bl):
    src = edge_index[0].astype(jnp.int32)
    dst = edge_index[1].astype(jnp.int32)
    pad = EPAD - NE
    srcp = jnp.concatenate([src, jnp.zeros((pad,), jnp.int32)])
    srcp = srcp.reshape(NS, CPS, CHUNK)
    dstp = jnp.concatenate([dst, jnp.full((pad,), NN, jnp.int32)])
    dstp = dstp.reshape(NS, CPS, CHUNK)

    srcl, dstl, cnt = _part_kernel(srcp, dstp)
    degp = _deg_kernel(dstl, cnt)
    g1, dinv = _tc1(x, W1, degp)
    p1 = _prop_kernel(g1, srcl, dstl, cnt)
    g2 = _tc_mid(p1, g1, dinv, b1.reshape(1, F), W2)
    p2 = _prop_kernel(g2, srcl, dstl, cnt)
    g3 = _tc_mid(p2, g2, dinv, b2.reshape(1, F), W3)
    p3 = _prop_kernel(g3, srcl, dstl, cnt)

    wlp = jnp.pad(Wl, ((0, 0), (0, F - NCLS)))
    blp = jnp.pad(bl, (0, F - NCLS)).reshape(1, F)
    out = _tc_fin(p3, g3, dinv, b3.reshape(1, F),
                  batch.astype(jnp.int32).reshape(1, NN), wlp, blp)
    return out[:, :NCLS]
